# Initial kernel scaffold; baseline (speedup 1.0000x reference)
import jax
import jax.numpy as jnp
from jax.experimental import pallas as pl


def kernel(input, boxes):
    raise NotImplementedError("write your pallas kernel here")


# speedup vs baseline: 178.2356x; 178.2356x over previous
import functools

import jax
import jax.numpy as jnp
from jax import lax
from jax.experimental import pallas as pl
from jax.experimental.pallas import tpu as pltpu
from jax.experimental.pallas import tpu_sc as plsc

_POOL = 10
_P = _POOL + 1
_N, _C4, _H, _W = 2, 128, 80, 100
_K = _H * _W
_C = _C4 // 4
_CPT = 8
_NBLK = _C4 // _CPT
_NW = 32
_CHUNK = 2000
_NCHUNK = _K // _CHUNK
_G = 16
_NGRP = _CHUNK // _G

_mesh = plsc.VectorSubcoreMesh(core_axis_name="c", subcore_axis_name="s")


@functools.partial(
    pl.kernel,
    out_type=jax.ShapeDtypeStruct((_NW, _NCHUNK, _CPT, _CHUNK), jnp.float32),
    mesh=_mesh,
    compiler_params=pltpu.CompilerParams(
        needs_layout_passes=False, use_tc_tiling_on_sc=False),
    scratch_types=[
        pltpu.VMEM((_CPT * _K,), jnp.float32),
        pltpu.VMEM((4, _K), jnp.float32),
        pltpu.VMEM((_CPT, _CHUNK), jnp.float32),
    ],
)
def _border_align_sc(inp_hbm, boxes_hbm, out_hbm, slab_v, box_v, outc_v):
    wid = lax.axis_index("s") * 2 + lax.axis_index("c")
    n = wid // _NBLK
    blk = wid % _NBLK
    border = blk // 4
    c0 = blk * _CPT

    for ci in range(_CPT):
        pltpu.sync_copy(inp_hbm.at[n, c0 + ci, :], slab_v.at[pl.ds(ci * _K, _K)])
    pltpu.sync_copy(boxes_hbm.at[n], box_v)

    bsel = jnp.where(border >= 2, jnp.float32(1.0), jnp.float32(0.0))
    ax = (jnp.where(border == 0, jnp.float32(1.0), jnp.float32(0.0))
          - jnp.where(border == 2, jnp.float32(1.0), jnp.float32(0.0)))
    ay = (jnp.where(border == 1, jnp.float32(1.0), jnp.float32(0.0))
          - jnp.where(border == 3, jnp.float32(1.0), jnp.float32(0.0)))

    def do_group(g, chunk):
        kb = chunk * _CHUNK + g * _G
        x1 = box_v[0, pl.ds(kb, _G)]
        y1 = box_v[1, pl.ds(kb, _G)]
        x2 = box_v[2, pl.ds(kb, _G)]
        y2 = box_v[3, pl.ds(kb, _G)]
        wx = x2 - x1
        wy = y2 - y1
        dx = wx * (ax * (1.0 / _POOL))
        dy = wy * (ay * (1.0 / _POOL))
        x0 = x1 + wx * bsel
        y0 = y1 + wy * bsel
        m = [None] * _CPT
        for p in range(_P):
            x = jnp.maximum(x0 + jnp.float32(p) * dx, 0.0)
            y = jnp.maximum(y0 + jnp.float32(p) * dy, 0.0)
            xl = x.astype(jnp.int32)
            yl = y.astype(jnp.int32)
            xh = jnp.minimum(xl + 1, _W - 1)
            yh = jnp.minimum(yl + 1, _H - 1)
            lx = jnp.where(xl >= _W - 1, jnp.float32(_W - 1), x) - xl.astype(jnp.float32)
            ly = jnp.where(yl >= _H - 1, jnp.float32(_H - 1), y) - yl.astype(jnp.float32)
            hx = 1.0 - lx
            hy = 1.0 - ly
            w11 = hy * hx
            w12 = hy * lx
            w21 = ly * hx
            w22 = ly * lx
            rl = yl * _W
            rh = yh * _W
            i11 = rl + xl
            i12 = rl + xh
            i21 = rh + xl
            i22 = rh + xh
            for ci in range(_CPT):
                off = ci * _K
                g11 = plsc.load_gather(slab_v, [i11 + off])
                g12 = plsc.load_gather(slab_v, [i12 + off])
                g21 = plsc.load_gather(slab_v, [i21 + off])
                g22 = plsc.load_gather(slab_v, [i22 + off])
                v = w11 * g11 + w12 * g12 + w21 * g21 + w22 * g22
                m[ci] = v if m[ci] is None else jnp.maximum(m[ci], v)
        for ci in range(_CPT):
            outc_v[ci, pl.ds(g * _G, _G)] = m[ci]
        return chunk

    def do_chunk(chunk, carry):
        lax.fori_loop(0, _NGRP, do_group, chunk)
        pltpu.sync_copy(outc_v, out_hbm.at[wid, chunk])
        return carry

    lax.fori_loop(0, _NCHUNK, do_chunk, 0)


def kernel(input, boxes):
    inp_r = input.reshape(_N, _C4, _K)
    boxes_t = boxes.transpose(0, 2, 1)
    o = _border_align_sc(inp_r, boxes_t)
    o = o.reshape(_N, 4, _C // _CPT, _NCHUNK, _CPT, _CHUNK)
    o = o.transpose(0, 2, 4, 3, 5, 1)
    return o.reshape(_N, _C, _K, 4)


# speedup vs baseline: 220.1817x; 1.2353x over previous
import functools

import jax
import jax.numpy as jnp
from jax import lax
from jax.experimental import pallas as pl
from jax.experimental.pallas import tpu as pltpu
from jax.experimental.pallas import tpu_sc as plsc

_POOL = 10
_P = _POOL + 1
_N, _C4, _H, _W = 2, 128, 80, 100
_K = _H * _W
_C = _C4 // 4
_CPT = 8
_PPT = _CPT // 2
_NBLK = _C4 // _CPT
_NW = 32
_CHUNK = 2000
_NCHUNK = _K // _CHUNK
_G = 16
_NGRP = _CHUNK // _G

_mesh = plsc.VectorSubcoreMesh(core_axis_name="c", subcore_axis_name="s")
_ILV = plsc.PackFormat.INTERLEAVED


@functools.partial(
    pl.kernel,
    out_type=jax.ShapeDtypeStruct((_NW, _NCHUNK, _CPT, _CHUNK), jnp.float32),
    mesh=_mesh,
    compiler_params=pltpu.CompilerParams(
        needs_layout_passes=False, use_tc_tiling_on_sc=False),
    scratch_types=[
        pltpu.VMEM((_PPT * _K,), jnp.float32),
        pltpu.VMEM((4, _K), jnp.float32),
        pltpu.VMEM((_CPT, _CHUNK), jnp.float32),
    ],
)
def _border_align_sc(inp_hbm, boxes_hbm, out_hbm, slab_v, box_v, outc_v):
    wid = lax.axis_index("s") * 2 + lax.axis_index("c")
    n = wid // _NBLK
    blk = wid % _NBLK
    border = blk // 4
    p0 = blk * _PPT

    for pr in range(_PPT):
        pltpu.sync_copy(inp_hbm.at[n, p0 + pr, :], slab_v.at[pl.ds(pr * _K, _K)])
    pltpu.sync_copy(boxes_hbm.at[n], box_v)

    bsel = jnp.where(border >= 2, jnp.float32(1.0), jnp.float32(0.0))
    ax = (jnp.where(border == 0, jnp.float32(1.0), jnp.float32(0.0))
          - jnp.where(border == 2, jnp.float32(1.0), jnp.float32(0.0)))
    ay = (jnp.where(border == 1, jnp.float32(1.0), jnp.float32(0.0))
          - jnp.where(border == 3, jnp.float32(1.0), jnp.float32(0.0)))

    def do_group(g, chunk):
        kb = chunk * _CHUNK + g * _G
        x1 = box_v[0, pl.ds(kb, _G)]
        y1 = box_v[1, pl.ds(kb, _G)]
        x2 = box_v[2, pl.ds(kb, _G)]
        y2 = box_v[3, pl.ds(kb, _G)]
        wx = x2 - x1
        wy = y2 - y1
        dx = wx * (ax * (1.0 / _POOL))
        dy = wy * (ay * (1.0 / _POOL))
        x0 = x1 + wx * bsel
        y0 = y1 + wy * bsel
        m = [None] * _PPT
        for p in range(_P):
            x = jnp.maximum(x0 + jnp.float32(p) * dx, 0.0)
            y = jnp.maximum(y0 + jnp.float32(p) * dy, 0.0)
            xl = x.astype(jnp.int32)
            yl = y.astype(jnp.int32)
            xh = jnp.minimum(xl + 1, _W - 1)
            yh = jnp.minimum(yl + 1, _H - 1)
            lx = jnp.where(xl >= _W - 1, jnp.float32(_W - 1), x) - xl.astype(jnp.float32)
            ly = jnp.where(yl >= _H - 1, jnp.float32(_H - 1), y) - yl.astype(jnp.float32)
            hx = 1.0 - lx
            hy = 1.0 - ly
            w11 = plsc.pack(hy * hx, hy * hx, format=_ILV)
            w12 = plsc.pack(hy * lx, hy * lx, format=_ILV)
            w21 = plsc.pack(ly * hx, ly * hx, format=_ILV)
            w22 = plsc.pack(ly * lx, ly * lx, format=_ILV)
            rl = yl * _W
            rh = yh * _W
            i11 = rl + xl
            i12 = rl + xh
            i21 = rh + xl
            i22 = rh + xh
            for pr in range(_PPT):
                off = pr * _K
                g11 = plsc.bitcast(plsc.load_gather(slab_v, [i11 + off]), jnp.bfloat16)
                g12 = plsc.bitcast(plsc.load_gather(slab_v, [i12 + off]), jnp.bfloat16)
                g21 = plsc.bitcast(plsc.load_gather(slab_v, [i21 + off]), jnp.bfloat16)
                g22 = plsc.bitcast(plsc.load_gather(slab_v, [i22 + off]), jnp.bfloat16)
                v = w11 * g11 + w12 * g12 + w21 * g21 + w22 * g22
                m[pr] = v if m[pr] is None else jnp.maximum(m[pr], v)
        for pr in range(_PPT):
            a, b = plsc.unpack(m[pr], format=_ILV)
            outc_v[2 * pr, pl.ds(g * _G, _G)] = a.astype(jnp.float32)
            outc_v[2 * pr + 1, pl.ds(g * _G, _G)] = b.astype(jnp.float32)
        return chunk

    def do_chunk(chunk, carry):
        lax.fori_loop(0, _NGRP, do_group, chunk)
        pltpu.sync_copy(outc_v, out_hbm.at[wid, chunk])
        return carry

    lax.fori_loop(0, _NCHUNK, do_chunk, 0)


def kernel(input, boxes):
    xbf = input.reshape(_N, _C4, _K).astype(jnp.bfloat16)
    packed = lax.bitcast_convert_type(
        xbf.reshape(_N, _C4 // 2, 2, _K).transpose(0, 1, 3, 2), jnp.float32)
    boxes_t = boxes.transpose(0, 2, 1)
    o = _border_align_sc(packed, boxes_t)
    o = o.reshape(_N, 4, _C // _CPT, _NCHUNK, _CPT, _CHUNK)
    o = o.transpose(0, 2, 4, 3, 5, 1)
    return o.reshape(_N, _C, _K, 4)


# speedup vs baseline: 227.6361x; 1.0339x over previous
import functools

import jax
import jax.numpy as jnp
from jax import lax
from jax.experimental import pallas as pl
from jax.experimental.pallas import tpu as pltpu
from jax.experimental.pallas import tpu_sc as plsc

_POOL = 10
_P = _POOL + 1
_N, _C4, _H, _W = 2, 128, 80, 100
_K = _H * _W
_C = _C4 // 4
_CPT = 8
_PPT = _CPT // 2
_NBLK = _C4 // _CPT
_NW = 32
_CHUNK = 2000
_NCHUNK = _K // _CHUNK
_G = 16
_NGRP = _CHUNK // _G

_mesh = plsc.VectorSubcoreMesh(core_axis_name="c", subcore_axis_name="s")
_ILV = plsc.PackFormat.INTERLEAVED


@functools.partial(
    pl.kernel,
    out_type=jax.ShapeDtypeStruct((_NW * _NCHUNK * _CPT * _CHUNK,), jnp.float32),
    mesh=_mesh,
    compiler_params=pltpu.CompilerParams(
        needs_layout_passes=False, use_tc_tiling_on_sc=False),
    scratch_types=[
        [pltpu.VMEM((_K,), jnp.float32)] * _PPT,
        pltpu.VMEM((4 * _K,), jnp.float32),
        pltpu.VMEM((_CPT * _CHUNK,), jnp.float32),
    ],
)
def _border_align_sc(inp_hbm, boxes_hbm, out_hbm, slabs_v, box_v, outc_v):
    wid = lax.axis_index("s") * 2 + lax.axis_index("c")
    n = wid // _NBLK
    blk = wid % _NBLK
    border = blk // 4
    p0 = blk * _PPT

    for pr in range(_PPT):
        pltpu.sync_copy(inp_hbm.at[pl.ds((n * (_C4 // 2) + p0 + pr) * _K, _K)],
                        slabs_v[pr])
    pltpu.sync_copy(boxes_hbm.at[pl.ds(n * 4 * _K, 4 * _K)], box_v)

    bsel = jnp.where(border >= 2, jnp.float32(1.0), jnp.float32(0.0))
    ax = (jnp.where(border == 0, jnp.float32(1.0), jnp.float32(0.0))
          - jnp.where(border == 2, jnp.float32(1.0), jnp.float32(0.0)))
    ay = (jnp.where(border == 1, jnp.float32(1.0), jnp.float32(0.0))
          - jnp.where(border == 3, jnp.float32(1.0), jnp.float32(0.0)))

    def do_group(g, chunk):
        kb = chunk * _CHUNK + g * _G
        x1 = box_v[pl.ds(kb, _G)]
        y1 = box_v[pl.ds(_K + kb, _G)]
        x2 = box_v[pl.ds(2 * _K + kb, _G)]
        y2 = box_v[pl.ds(3 * _K + kb, _G)]
        wx = x2 - x1
        wy = y2 - y1
        dx = wx * (ax * (1.0 / _POOL))
        dy = wy * (ay * (1.0 / _POOL))
        x0 = x1 + wx * bsel
        y0 = y1 + wy * bsel
        m = [None] * _PPT
        for p in range(_P):
            x = jnp.maximum(x0 + jnp.float32(p) * dx, 0.0)
            y = jnp.maximum(y0 + jnp.float32(p) * dy, 0.0)
            xl = x.astype(jnp.int32)
            yl = y.astype(jnp.int32)
            xh = jnp.minimum(xl + 1, _W - 1)
            yh = jnp.minimum(yl + 1, _H - 1)
            lx = jnp.where(xl >= _W - 1, jnp.float32(_W - 1), x) - xl.astype(jnp.float32)
            ly = jnp.where(yl >= _H - 1, jnp.float32(_H - 1), y) - yl.astype(jnp.float32)
            hx = 1.0 - lx
            hy = 1.0 - ly
            w11 = plsc.pack(hy * hx, hy * hx, format=_ILV)
            w12 = plsc.pack(hy * lx, hy * lx, format=_ILV)
            w21 = plsc.pack(ly * hx, ly * hx, format=_ILV)
            w22 = plsc.pack(ly * lx, ly * lx, format=_ILV)
            rl = yl * _W
            rh = yh * _W
            i11 = rl + xl
            i12 = rl + xh
            i21 = rh + xl
            i22 = rh + xh
            for pr in range(_PPT):
                sl = slabs_v[pr]
                g11 = plsc.bitcast(plsc.load_gather(sl, [i11]), jnp.bfloat16)
                g12 = plsc.bitcast(plsc.load_gather(sl, [i12]), jnp.bfloat16)
                g21 = plsc.bitcast(plsc.load_gather(sl, [i21]), jnp.bfloat16)
                g22 = plsc.bitcast(plsc.load_gather(sl, [i22]), jnp.bfloat16)
                v = w11 * g11 + w12 * g12 + w21 * g21 + w22 * g22
                m[pr] = v if m[pr] is None else jnp.maximum(m[pr], v)
        for pr in range(_PPT):
            a, b = plsc.unpack(m[pr], format=_ILV)
            outc_v[pl.ds(2 * pr * _CHUNK + g * _G, _G)] = a.astype(jnp.float32)
            outc_v[pl.ds((2 * pr + 1) * _CHUNK + g * _G, _G)] = b.astype(jnp.float32)
        return chunk

    def do_chunk(chunk, carry):
        lax.fori_loop(0, _NGRP, do_group, chunk)
        pltpu.sync_copy(
            outc_v,
            out_hbm.at[pl.ds((wid * _NCHUNK + chunk) * _CPT * _CHUNK,
                             _CPT * _CHUNK)])
        return carry

    lax.fori_loop(0, _NCHUNK, do_chunk, 0)


def kernel(input, boxes):
    xbf = input.reshape(_N, _C4, _K).astype(jnp.bfloat16)
    packed = lax.bitcast_convert_type(
        xbf.reshape(_N, _C4 // 2, 2, _K).transpose(0, 1, 3, 2), jnp.float32)
    boxes_t = boxes.transpose(0, 2, 1)
    o = _border_align_sc(packed.reshape(-1), boxes_t.reshape(-1))
    o = o.reshape(_N, 4, _C // _CPT, _NCHUNK, _CPT, _CHUNK)
    o = o.transpose(0, 2, 4, 3, 5, 1)
    return o.reshape(_N, _C, _K, 4)


# speedup vs baseline: 239.7858x; 1.0534x over previous
import functools

import jax
import jax.numpy as jnp
from jax import lax
from jax.experimental import pallas as pl
from jax.experimental.pallas import tpu as pltpu
from jax.experimental.pallas import tpu_sc as plsc

_POOL = 10
_P = _POOL + 1
_N, _C4, _H, _W = 2, 128, 80, 100
_K = _H * _W
_C = _C4 // 4
_CPT = 8
_PPT = _CPT // 2
_NBLK = _C4 // _CPT
_NW = 32
_CHUNK = 2000
_NCHUNK = _K // _CHUNK
_G = 16
_NGRP = _CHUNK // _G

_mesh = plsc.VectorSubcoreMesh(core_axis_name="c", subcore_axis_name="s")
_ILV = plsc.PackFormat.INTERLEAVED


@functools.partial(
    pl.kernel,
    out_type=jax.ShapeDtypeStruct((_N * 4 * _C, _K), jnp.float32),
    mesh=_mesh,
    compiler_params=pltpu.CompilerParams(
        needs_layout_passes=False, use_tc_tiling_on_sc=False),
    scratch_types=[
        [pltpu.VMEM((_K,), jnp.float32)] * _PPT,
        pltpu.VMEM((4 * _K,), jnp.float32),
        pltpu.VMEM((_CPT, _CHUNK), jnp.float32),
    ],
)
def _border_align_sc(inp_hbm, boxes_hbm, out_hbm, slabs_v, box_v, outc_v):
    wid = lax.axis_index("s") * 2 + lax.axis_index("c")
    n = wid // _NBLK
    blk = wid % _NBLK
    border = blk // 4
    p0 = blk * _PPT

    for pr in range(_PPT):
        pltpu.sync_copy(inp_hbm.at[pl.ds((n * (_C4 // 2) + p0 + pr) * _K, _K)],
                        slabs_v[pr])
    pltpu.sync_copy(boxes_hbm.at[pl.ds(n * 4 * _K, 4 * _K)], box_v)

    bsel = jnp.where(border >= 2, jnp.float32(1.0), jnp.float32(0.0))
    ax = (jnp.where(border == 0, jnp.float32(1.0), jnp.float32(0.0))
          - jnp.where(border == 2, jnp.float32(1.0), jnp.float32(0.0)))
    ay = (jnp.where(border == 1, jnp.float32(1.0), jnp.float32(0.0))
          - jnp.where(border == 3, jnp.float32(1.0), jnp.float32(0.0)))

    def do_group(g, chunk):
        kb = chunk * _CHUNK + g * _G
        x1 = box_v[pl.ds(kb, _G)]
        y1 = box_v[pl.ds(_K + kb, _G)]
        x2 = box_v[pl.ds(2 * _K + kb, _G)]
        y2 = box_v[pl.ds(3 * _K + kb, _G)]
        wx = x2 - x1
        wy = y2 - y1
        dx = wx * (ax * (1.0 / _POOL))
        dy = wy * (ay * (1.0 / _POOL))
        x0 = x1 + wx * bsel
        y0 = y1 + wy * bsel
        m = [None] * _PPT
        for p in range(_P):
            x = jnp.maximum(x0 + jnp.float32(p) * dx, 0.0)
            y = jnp.maximum(y0 + jnp.float32(p) * dy, 0.0)
            xl = x.astype(jnp.int32)
            yl = y.astype(jnp.int32)
            xh = jnp.minimum(xl + 1, _W - 1)
            yh = jnp.minimum(yl + 1, _H - 1)
            lx = jnp.where(xl >= _W - 1, jnp.float32(_W - 1), x) - xl.astype(jnp.float32)
            ly = jnp.where(yl >= _H - 1, jnp.float32(_H - 1), y) - yl.astype(jnp.float32)
            hx = 1.0 - lx
            hy = 1.0 - ly
            w11 = plsc.pack(hy * hx, hy * hx, format=_ILV)
            w12 = plsc.pack(hy * lx, hy * lx, format=_ILV)
            w21 = plsc.pack(ly * hx, ly * hx, format=_ILV)
            w22 = plsc.pack(ly * lx, ly * lx, format=_ILV)
            rl = yl * _W
            rh = yh * _W
            i11 = rl + xl
            i12 = rl + xh
            i21 = rh + xl
            i22 = rh + xh
            for pr in range(_PPT):
                sl = slabs_v[pr]
                g11 = plsc.bitcast(plsc.load_gather(sl, [i11]), jnp.bfloat16)
                g12 = plsc.bitcast(plsc.load_gather(sl, [i12]), jnp.bfloat16)
                g21 = plsc.bitcast(plsc.load_gather(sl, [i21]), jnp.bfloat16)
                g22 = plsc.bitcast(plsc.load_gather(sl, [i22]), jnp.bfloat16)
                v = w11 * g11 + w12 * g12 + w21 * g21 + w22 * g22
                m[pr] = v if m[pr] is None else jnp.maximum(m[pr], v)
        for pr in range(_PPT):
            a, b = plsc.unpack(m[pr], format=_ILV)
            outc_v[2 * pr, pl.ds(g * _G, _G)] = a.astype(jnp.float32)
            outc_v[2 * pr + 1, pl.ds(g * _G, _G)] = b.astype(jnp.float32)
        return chunk

    def do_chunk(chunk, carry):
        lax.fori_loop(0, _NGRP, do_group, chunk)
        row0 = n * (4 * _C) + blk * _CPT
        pltpu.sync_copy(outc_v,
                        out_hbm.at[pl.ds(row0, _CPT),
                                   pl.ds(chunk * _CHUNK, _CHUNK)])
        return carry

    lax.fori_loop(0, _NCHUNK, do_chunk, 0)


def kernel(input, boxes):
    xbf = input.reshape(_N, _C4 // 2, 2, _K).astype(jnp.bfloat16)
    u = lax.bitcast_convert_type(xbf, jnp.uint16).astype(jnp.uint32)
    packed = lax.bitcast_convert_type(u[:, :, 0, :] | (u[:, :, 1, :] << 16),
                                      jnp.float32)
    boxes_t = boxes.transpose(0, 2, 1)
    o = _border_align_sc(packed.reshape(-1), boxes_t.reshape(-1))
    return o.reshape(_N, 4, _C, _K).transpose(0, 2, 3, 1)


# speedup vs baseline: 244.0502x; 1.0178x over previous
import functools

import jax
import jax.numpy as jnp
from jax import lax
from jax.experimental import pallas as pl
from jax.experimental.pallas import tpu as pltpu
from jax.experimental.pallas import tpu_sc as plsc

_POOL = 10
_P = _POOL + 1
_N, _C4, _H, _W = 2, 128, 80, 100
_K = _H * _W
_C = _C4 // 4
_CPT = 8
_PPT = _CPT // 2
_NBLK = _C4 // _CPT
_NW = 32
_CHUNK = 2000
_NCHUNK = _K // _CHUNK
_G = 16
_NGRP = _CHUNK // _G

_mesh = plsc.VectorSubcoreMesh(core_axis_name="c", subcore_axis_name="s")
_ILV = plsc.PackFormat.INTERLEAVED


@functools.partial(
    pl.kernel,
    out_type=jax.ShapeDtypeStruct((_N * 4 * _C, _K), jnp.float32),
    mesh=_mesh,
    compiler_params=pltpu.CompilerParams(
        needs_layout_passes=False, use_tc_tiling_on_sc=False),
    scratch_types=[
        [pltpu.VMEM((_K,), jnp.float32)] * _PPT,
        [pltpu.VMEM((_H, _W), jnp.float32)] * 2,
        pltpu.VMEM((4, _K), jnp.float32),
        pltpu.VMEM((_CPT, _CHUNK), jnp.float32),
    ],
)
def _border_align_sc(inp_hbm, boxes_hbm, out_hbm, slabs_v, planes_v, box_v,
                     outc_v):
    wid = lax.axis_index("s") * 2 + lax.axis_index("c")
    n = wid // _NBLK
    blk = wid % _NBLK
    border = blk // 4
    c0 = blk * _CPT

    _COLS = tuple(range(0, _W - _G + 1, _G)) + (_W - _G,)
    for pr in range(_PPT):
        pltpu.sync_copy(inp_hbm.at[n, c0 + 2 * pr], planes_v[0])
        pltpu.sync_copy(inp_hbm.at[n, c0 + 2 * pr + 1], planes_v[1])
        slab = slabs_v[pr]

        def pack_row(r, _, slab=slab):
            base = r * _W
            for col in _COLS:
                a = planes_v[0][r, pl.ds(col, _G)]
                b = planes_v[1][r, pl.ds(col, _G)]
                slab[pl.ds(base + col, _G)] = plsc.bitcast(
                    plsc.pack(a, b, format=_ILV), jnp.float32)
            return 0

        lax.fori_loop(0, _H, pack_row, 0)
    pltpu.sync_copy(boxes_hbm.at[n], box_v)

    bsel = jnp.where(border >= 2, jnp.float32(1.0), jnp.float32(0.0))
    ax = (jnp.where(border == 0, jnp.float32(1.0), jnp.float32(0.0))
          - jnp.where(border == 2, jnp.float32(1.0), jnp.float32(0.0)))
    ay = (jnp.where(border == 1, jnp.float32(1.0), jnp.float32(0.0))
          - jnp.where(border == 3, jnp.float32(1.0), jnp.float32(0.0)))

    def do_group(g, chunk):
        kb = chunk * _CHUNK + g * _G
        x1 = box_v[0, pl.ds(kb, _G)]
        y1 = box_v[1, pl.ds(kb, _G)]
        x2 = box_v[2, pl.ds(kb, _G)]
        y2 = box_v[3, pl.ds(kb, _G)]
        wx = x2 - x1
        wy = y2 - y1
        dx = wx * (ax * (1.0 / _POOL))
        dy = wy * (ay * (1.0 / _POOL))
        x0 = x1 + wx * bsel
        y0 = y1 + wy * bsel
        m = [None] * _PPT
        for p in range(_P):
            x = jnp.maximum(x0 + jnp.float32(p) * dx, 0.0)
            y = jnp.maximum(y0 + jnp.float32(p) * dy, 0.0)
            xl = x.astype(jnp.int32)
            yl = y.astype(jnp.int32)
            xh = jnp.minimum(xl + 1, _W - 1)
            yh = jnp.minimum(yl + 1, _H - 1)
            lx = jnp.where(xl >= _W - 1, jnp.float32(_W - 1), x) - xl.astype(jnp.float32)
            ly = jnp.where(yl >= _H - 1, jnp.float32(_H - 1), y) - yl.astype(jnp.float32)
            hx = 1.0 - lx
            hy = 1.0 - ly
            w11 = plsc.pack(hy * hx, hy * hx, format=_ILV)
            w12 = plsc.pack(hy * lx, hy * lx, format=_ILV)
            w21 = plsc.pack(ly * hx, ly * hx, format=_ILV)
            w22 = plsc.pack(ly * lx, ly * lx, format=_ILV)
            rl = yl * _W
            rh = yh * _W
            i11 = rl + xl
            i12 = rl + xh
            i21 = rh + xl
            i22 = rh + xh
            for pr in range(_PPT):
                sl = slabs_v[pr]
                g11 = plsc.bitcast(plsc.load_gather(sl, [i11]), jnp.bfloat16)
                g12 = plsc.bitcast(plsc.load_gather(sl, [i12]), jnp.bfloat16)
                g21 = plsc.bitcast(plsc.load_gather(sl, [i21]), jnp.bfloat16)
                g22 = plsc.bitcast(plsc.load_gather(sl, [i22]), jnp.bfloat16)
                v = w11 * g11 + w12 * g12 + w21 * g21 + w22 * g22
                m[pr] = v if m[pr] is None else jnp.maximum(m[pr], v)
        for pr in range(_PPT):
            a, b = plsc.unpack(m[pr], format=_ILV)
            outc_v[2 * pr, pl.ds(g * _G, _G)] = a.astype(jnp.float32)
            outc_v[2 * pr + 1, pl.ds(g * _G, _G)] = b.astype(jnp.float32)
        return chunk

    def do_chunk(chunk, carry):
        lax.fori_loop(0, _NGRP, do_group, chunk)
        row0 = n * (4 * _C) + blk * _CPT
        pltpu.sync_copy(outc_v,
                        out_hbm.at[pl.ds(row0, _CPT),
                                   pl.ds(chunk * _CHUNK, _CHUNK)])
        return carry

    lax.fori_loop(0, _NCHUNK, do_chunk, 0)


def kernel(input, boxes):
    boxes_t = boxes.transpose(0, 2, 1)
    o = _border_align_sc(input, boxes_t)
    return o.reshape(_N, 4, _C, _K).transpose(0, 2, 3, 1)
